# baseline (speedup 1.0000x reference)
import functools

import jax
import jax.numpy as jnp
from jax import lax
from jax.experimental import pallas as pl
from jax.experimental.pallas import tpu as pltpu
from jax.experimental.pallas import tpu_sc as plsc

B_ = 1024
L_ = 200
H_ = 768
NTOK = B_ * L_
NW = 32
HALF = NTOK // 2
TOK_PW = HALF // NW
TG = 64
NITG = TOK_PW // TG
BT = 1600
NBLK = HALF // BT
LN_EPS_ = 1e-12


def _sc_gather(ids, word):
    mesh = plsc.VectorSubcoreMesh(core_axis_name="c", subcore_axis_name="s")

    @functools.partial(
        pl.kernel,
        mesh=mesh,
        compiler_params=pltpu.CompilerParams(needs_layout_passes=False),
        out_type=jax.ShapeDtypeStruct((HALF, H_), jnp.float32),
        scratch_types=[
            pltpu.VMEM((2, TG, H_), jnp.float32),
            pltpu.VMEM((2, TG), jnp.int32),
            pltpu.SemaphoreType.DMA,
            pltpu.SemaphoreType.DMA,
            pltpu.SemaphoreType.DMA,
            pltpu.SemaphoreType.DMA,
        ],
    )
    def k(ids_h, word_h, out_h, rowsb, idxb, g0, g1, o0, o1):
        cid = lax.axis_index("c")
        sid = lax.axis_index("s")
        wid = sid * 2 + cid
        base = wid * TOK_PW
        gsem = (g0, g1)
        osem = (o0, o1)

        def start_gather(i, k_):
            pltpu.sync_copy(ids_h.at[pl.ds(base + i * TG, TG)], idxb.at[k_])
            pltpu.async_copy(word_h.at[idxb.at[k_]], rowsb.at[k_], gsem[k_])

        def wait_gather(k_):
            pltpu.make_async_copy(word_h.at[idxb.at[k_]], rowsb.at[k_],
                                  gsem[k_]).wait()

        def start_scatter(i, k_):
            pltpu.async_copy(rowsb.at[k_],
                             out_h.at[pl.ds(base + i * TG, TG)], osem[k_])

        def wait_scatter(k_):
            pltpu.make_async_copy(rowsb.at[k_], out_h.at[pl.ds(0, TG)],
                                  osem[k_]).wait()

        start_gather(0, 0)

        def outer(it, carry):
            i0 = it * 2
            for kb in range(2):
                i = i0 + kb
                nk = 1 - kb

                @pl.when(i + 1 < NITG)
                def _():
                    @pl.when(i >= 1)
                    def _():
                        wait_scatter(nk)
                    start_gather(i + 1, nk)

                wait_gather(kb)
                start_scatter(i, kb)
            return carry

        lax.fori_loop(0, NITG // 2, outer, 0)
        wait_scatter(0)
        wait_scatter(1)

    return k(ids, word)


def _tc_math(st_ref, sg_ref, pos_ref, c_ref, o_ref):
    sf = sg_ref[...].astype(jnp.float32)
    x = st_ref[...] + c_ref[0:1, :] + sf * c_ref[1:2, :]
    p = pos_ref[...]
    x = x + jnp.concatenate([p] * (BT // L_), axis=0)
    mu = jnp.mean(x, axis=-1, keepdims=True)
    xc = x - mu
    var = jnp.mean(xc * xc, axis=-1, keepdims=True)
    y = xc * lax.rsqrt(var + LN_EPS_) * c_ref[4:5, :]
    o_ref[...] = y + c_ref[2:3, :] + sf * c_ref[3:4, :]


def _tc_body1(st_ref, sg_ref, pos_ref, c_ref, o_ref):
    _tc_math(st_ref, sg_ref, pos_ref, c_ref, o_ref)


def _tc_body2(st_ref, sg_ref, pos_ref, c_ref, carry_ref, o_ref):
    del carry_ref
    _tc_math(st_ref, sg_ref, pos_ref, c_ref, o_ref)


_HALF_SPECS = [
    pl.BlockSpec((BT, H_), lambda b: (b, 0)),
    pl.BlockSpec((BT, 1), lambda b: (b, 0)),
    pl.BlockSpec((L_, H_), lambda b: (0, 0)),
    pl.BlockSpec((5, H_), lambda b: (0, 0)),
]


def _tc_ln_first(staged, seg2d, pos200, consts):
    return pl.pallas_call(
        _tc_body1,
        grid=(NBLK,),
        in_specs=_HALF_SPECS,
        out_specs=pl.BlockSpec((BT, H_), lambda b: (b, 0)),
        out_shape=jax.ShapeDtypeStruct((NTOK, H_), jnp.float32),
        compiler_params=pltpu.CompilerParams(
            dimension_semantics=("arbitrary",)),
    )(staged, seg2d, pos200, consts)


def _tc_ln_second(staged, seg2d, pos200, consts, carry):
    return pl.pallas_call(
        _tc_body2,
        grid=(NBLK,),
        in_specs=_HALF_SPECS + [pl.BlockSpec(memory_space=pl.ANY)],
        out_specs=pl.BlockSpec((BT, H_), lambda b: (b + NBLK, 0)),
        out_shape=jax.ShapeDtypeStruct((NTOK, H_), jnp.float32),
        input_output_aliases={4: 0},
        compiler_params=pltpu.CompilerParams(
            dimension_semantics=("arbitrary",)),
    )(staged, seg2d, pos200, consts, carry)


def kernel(input_ids, segment_ids, word_emb, pos_emb, type_emb, ln_gamma,
           ln_beta, vilt_type_emb):
    ids = input_ids.reshape(-1)
    seg2d = segment_ids.reshape(-1, 1)

    staged1 = _sc_gather(ids[:HALF], word_emb)
    staged2 = _sc_gather(ids[HALF:], word_emb)

    pos200 = pos_emb[:L_]
    consts = jnp.stack([
        type_emb[0],
        type_emb[1] - type_emb[0],
        ln_beta + vilt_type_emb[0],
        vilt_type_emb[1] - vilt_type_emb[0],
        ln_gamma,
    ], axis=0)

    o1 = _tc_ln_first(staged1, seg2d[:HALF], pos200, consts)
    out = _tc_ln_second(staged2, seg2d[HALF:], pos200, consts, o1)
    return out.reshape(B_, L_, H_)


# speedup vs baseline: 1.0338x; 1.0338x over previous
import functools

import jax
import jax.numpy as jnp
from jax import lax
from jax.experimental import pallas as pl
from jax.experimental.pallas import tpu as pltpu
from jax.experimental.pallas import tpu_sc as plsc

B_ = 1024
L_ = 200
H_ = 768
NTOK = B_ * L_
NW = 32
TOK_PW = NTOK // NW
TG = 80
NITG = TOK_PW // TG
BT = 1600
NBLK = NTOK // BT
LN_EPS_ = 1e-12


def _sc_gather(ids, word):
    mesh = plsc.VectorSubcoreMesh(core_axis_name="c", subcore_axis_name="s")

    @functools.partial(
        pl.kernel,
        mesh=mesh,
        compiler_params=pltpu.CompilerParams(needs_layout_passes=False),
        out_type=jax.ShapeDtypeStruct((NTOK, H_), jnp.float32),
        scratch_types=[
            pltpu.VMEM((2, TG, H_), jnp.float32),
            pltpu.VMEM((2, TG), jnp.int32),
            pltpu.SemaphoreType.DMA,
            pltpu.SemaphoreType.DMA,
            pltpu.SemaphoreType.DMA,
            pltpu.SemaphoreType.DMA,
        ],
    )
    def k(ids_h, word_h, out_h, rowsb, idxb, g0, g1, o0, o1):
        cid = lax.axis_index("c")
        sid = lax.axis_index("s")
        wid = sid * 2 + cid
        base = wid * TOK_PW
        gsem = (g0, g1)
        osem = (o0, o1)

        def start_gather(i, k_):
            pltpu.sync_copy(ids_h.at[pl.ds(base + i * TG, TG)], idxb.at[k_])
            pltpu.async_copy(word_h.at[idxb.at[k_]], rowsb.at[k_], gsem[k_])

        def wait_gather(k_):
            pltpu.make_async_copy(word_h.at[idxb.at[k_]], rowsb.at[k_],
                                  gsem[k_]).wait()

        def start_scatter(i, k_):
            pltpu.async_copy(rowsb.at[k_],
                             out_h.at[pl.ds(base + i * TG, TG)], osem[k_])

        def wait_scatter(k_):
            pltpu.make_async_copy(rowsb.at[k_], out_h.at[pl.ds(0, TG)],
                                  osem[k_]).wait()

        start_gather(0, 0)

        def outer(it, carry):
            i0 = it * 2
            for kb in range(2):
                i = i0 + kb
                nk = 1 - kb

                @pl.when(i + 1 < NITG)
                def _():
                    @pl.when(i >= 1)
                    def _():
                        wait_scatter(nk)
                    start_gather(i + 1, nk)

                wait_gather(kb)
                start_scatter(i, kb)
            return carry

        lax.fori_loop(0, NITG // 2, outer, 0)
        wait_scatter(0)
        wait_scatter(1)

    return k(ids, word)


def _tc_math(st_ref, sg_ref, pos_ref, c_ref, o_ref):
    sf = sg_ref[...].astype(jnp.float32)
    x = st_ref[...] + c_ref[0:1, :] + sf * c_ref[1:2, :]
    p = pos_ref[...]
    x = x + jnp.concatenate([p] * (BT // L_), axis=0)
    mu = jnp.mean(x, axis=-1, keepdims=True)
    xc = x - mu
    var = jnp.mean(xc * xc, axis=-1, keepdims=True)
    y = xc * lax.rsqrt(var + LN_EPS_) * c_ref[4:5, :]
    o_ref[...] = y + c_ref[2:3, :] + sf * c_ref[3:4, :]


def _tc_ln(staged, seg2d, pos200, consts):
    return pl.pallas_call(
        _tc_math,
        grid=(NBLK,),
        in_specs=[
            pl.BlockSpec((BT, H_), lambda b: (b, 0)),
            pl.BlockSpec((BT, 1), lambda b: (b, 0)),
            pl.BlockSpec((L_, H_), lambda b: (0, 0)),
            pl.BlockSpec((5, H_), lambda b: (0, 0)),
        ],
        out_specs=pl.BlockSpec((BT, H_), lambda b: (b, 0)),
        out_shape=jax.ShapeDtypeStruct((NTOK, H_), jnp.float32),
        compiler_params=pltpu.CompilerParams(
            dimension_semantics=("arbitrary",)),
    )(staged, seg2d, pos200, consts)


def kernel(input_ids, segment_ids, word_emb, pos_emb, type_emb, ln_gamma,
           ln_beta, vilt_type_emb):
    ids = input_ids.reshape(-1)
    seg2d = segment_ids.reshape(-1, 1)

    staged = _sc_gather(ids, word_emb)

    pos200 = pos_emb[:L_]
    consts = jnp.stack([
        type_emb[0],
        type_emb[1] - type_emb[0],
        ln_beta + vilt_type_emb[0],
        vilt_type_emb[1] - vilt_type_emb[0],
        ln_gamma,
    ], axis=0)

    out = _tc_ln(staged, seg2d, pos200, consts)
    return out.reshape(B_, L_, H_)
